# baseline (speedup 1.0000x reference)
import functools

import numpy as np
import jax
import jax.numpy as jnp
from jax import lax
from jax.experimental import pallas as pl
from jax.experimental.pallas import tpu as pltpu
from jax.experimental.pallas import tpu_sc as plsc

_C, _H, _W = 3, 32, 32
_B, _D = 64, 256
_P = _C * _H * _W
_L = _P
_R = _L * _B


def _zigzag_gather_idx() -> np.ndarray:
    diagonals = [[] for _ in range(_H + _W - 1)]
    for i in range(_H):
        for j in range(_W):
            s = i + j
            if s % 2 == 0:
                diagonals[s].insert(0, (i, j))
            else:
                diagonals[s].append((i, j))
    pos = []
    for d in diagonals:
        for (i, j) in d:
            for c in range(_C):
                pos.append(c * _H * _W + i * _W + j)
    perm = np.asarray(pos, dtype=np.int64)
    r = np.arange(_R, dtype=np.int64)
    return ((r % _B) * _P + perm[r // _B]).astype(np.int32)


_GATHER_IDX = _zigzag_gather_idx()

_NW = 32
_ROWS_PER_W = _R // _NW
_K = 128
_CHUNKS = _ROWS_PER_W // _K


def _sc_gather(table, idx):
    mesh = plsc.VectorSubcoreMesh(core_axis_name="c", subcore_axis_name="s")

    @functools.partial(
        pl.kernel,
        mesh=mesh,
        out_type=jax.ShapeDtypeStruct((_R, _D), jnp.float32),
        scratch_types=[
            pltpu.VMEM((_ROWS_PER_W,), jnp.int32),
            pltpu.VMEM((_K, _D), jnp.float32),
            pltpu.VMEM((_K, _D), jnp.float32),
            pltpu.VMEM((_K, _D), jnp.float32),
            pltpu.SemaphoreType.DMA,
            pltpu.SemaphoreType.DMA,
            pltpu.SemaphoreType.DMA,
            pltpu.SemaphoreType.DMA,
            pltpu.SemaphoreType.DMA,
            pltpu.SemaphoreType.DMA,
        ],
    )
    def k(table_hbm, idx_hbm, out_hbm, idx_v, b0, b1, b2,
          g0, g1, g2, s0, s1, s2):
        bufs = (b0, b1, b2)
        gsem = (g0, g1, g2)
        ssem = (s0, s1, s2)
        wid = lax.axis_index("s") * 2 + lax.axis_index("c")
        base = wid * _ROWS_PER_W
        pltpu.sync_copy(idx_hbm.at[pl.ds(base, _ROWS_PER_W)], idx_v)

        def gather(chunk, b):
            return pltpu.make_async_copy(
                table_hbm.at[idx_v.at[pl.ds(chunk * _K, _K)]], bufs[b], gsem[b]
            )

        def scatter(chunk, b):
            return pltpu.make_async_copy(
                bufs[b], out_hbm.at[pl.ds(base + chunk * _K, _K)], ssem[b]
            )

        gather(0, 0).start()
        gather(1, 1).start()
        third = _CHUNKS // 3

        def body(t, carry):
            i0 = 3 * t
            for b in range(3):
                i = i0 + b
                bn = (b + 2) % 3
                gather(i, b).wait()
                scatter(i, b).start()

                if b == 0:
                    @pl.when(t > 0)
                    def _():
                        scatter(i - 1, bn).wait()

                    gather(i + 2, bn).start()
                else:
                    scatter(i - 1, bn).wait()

                    @pl.when(t < third - 1)
                    def _():
                        gather(i + 2, bn).start()
            return carry

        lax.fori_loop(0, third, body, 0)
        scatter(_CHUNKS - 1, 2).wait()

    return k(table, idx)


def kernel(x):
    table = x.reshape(_B * _P, _D)
    out = _sc_gather(table, jnp.asarray(_GATHER_IDX))
    return out.reshape(_L, _B, _D)
